# Initial kernel scaffold; baseline (speedup 1.0000x reference)
import jax
import jax.numpy as jnp
from jax.experimental import pallas as pl


def kernel(x, edge_index, edge_attr, We1, be1, We2, be2, Wc01, bc01, Wc02, bc02, Wc11, bc11, Wc12, bc12, Wd1, bd1, Wd2, bd2, Wd3, bd3):
    raise NotImplementedError("write your pallas kernel here")


# speedup vs baseline: 2.2802x; 2.2802x over previous
import jax
import jax.numpy as jnp
from jax import lax
from jax.experimental import pallas as pl
from jax.experimental.pallas import tpu as pltpu
from jax.experimental.pallas import tpu_sc as plsc

N = 10000
E = 320000
H = 32
F_IN = 128

NC = 2
NS = 16
NW = NC * NS

G_CHUNK = E // NW

S_EC = 8
S_CHUNK = E // S_EC
S_BLK = 400
S_NBLK = S_CHUNK // S_BLK
NH_SZ = N // 2
FH_SZ = H // 2

_f32 = jnp.float32
_i32 = jnp.int32

_SC_PARAMS = pltpu.CompilerParams(use_tc_tiling_on_sc=False,
                                  needs_layout_passes=False)


def _combine_body(parts_ref, h_ref):
    h_ref[...] = jnp.max(parts_ref[...], axis=0)


def _tc_combine(parts):
    blk = 1000
    return pl.pallas_call(
        _combine_body,
        grid=(N // blk,),
        in_specs=[pl.BlockSpec((S_EC, blk, H), lambda b: (0, b, 0))],
        out_specs=pl.BlockSpec((blk, H), lambda b: (b, 0)),
        out_shape=jax.ShapeDtypeStruct((N, H), _f32),
    )(parts)


def _edge_mlp(gi, gj, w1_ref, b1_ref, w2_ref, b2_ref):
    m = jnp.concatenate([gi, gj - gi], axis=-1)
    t = jax.nn.relu(jnp.dot(m, w1_ref[...], preferred_element_type=_f32)
                    + b1_ref[...])
    msg = jnp.dot(t, w2_ref[...], preferred_element_type=_f32) + b2_ref[...]
    return jnp.maximum(msg, 0.0)


def _msg_body(gi_ref, gj_ref, w1_ref, b1_ref, w2_ref, b2_ref, o_ref):
    o_ref[...] = _edge_mlp(gi_ref[...], gj_ref[...], w1_ref, b1_ref, w2_ref,
                           b2_ref)


def _tc_msg(gi, gj, w1_t, b1, w2_t, b2, f_in):
    blk = 2000
    return pl.pallas_call(
        _msg_body,
        grid=(E // blk,),
        in_specs=[
            pl.BlockSpec((blk, f_in), lambda b: (b, 0)),
            pl.BlockSpec((blk, f_in), lambda b: (b, 0)),
            pl.BlockSpec((2 * f_in, H), lambda b: (0, 0)),
            pl.BlockSpec((1, H), lambda b: (0, 0)),
            pl.BlockSpec((H, H), lambda b: (0, 0)),
            pl.BlockSpec((1, H), lambda b: (0, 0)),
        ],
        out_specs=pl.BlockSpec((blk, H), lambda b: (b, 0)),
        out_shape=jax.ShapeDtypeStruct((E, H), _f32),
    )(gi, gj, w1_t, b1.reshape(1, H), w2_t, b2.reshape(1, H))


def _tail_body(gi_ref, gj_ref, w1_ref, b1_ref, w2_ref, b2_ref, wd1_ref,
               bd1_ref, wd2_ref, bd2_ref, wd3_ref, bd3_ref, o_ref, zacc):
    i = pl.program_id(0)
    nsteps = pl.num_programs(0)
    m = _edge_mlp(gi_ref[...], gj_ref[...], w1_ref, b1_ref, w2_ref, b2_ref)
    bm = jnp.max(m.reshape(-1, 8, H), axis=0)

    @pl.when(i == 0)
    def _():
        zacc[...] = bm

    @pl.when(i > 0)
    def _():
        zacc[...] = jnp.maximum(zacc[...], bm)

    @pl.when(i == nsteps - 1)
    def _():
        z = jnp.max(zacc[...], axis=0, keepdims=True)
        d = jax.nn.relu(jnp.dot(z, wd1_ref[...], preferred_element_type=_f32)
                        + bd1_ref[...])
        d = jax.nn.relu(jnp.dot(d, wd2_ref[...], preferred_element_type=_f32)
                        + bd2_ref[...])
        o_ref[...] = (jnp.dot(d, wd3_ref[...], preferred_element_type=_f32)
                      + bd3_ref[...])


def _tc_tail(gi, gj, w1_t, b1, w2_t, b2, wd1_t, bd1, wd2_t, bd2, wd3_t, bd3):
    blk = 2000
    return pl.pallas_call(
        _tail_body,
        grid=(E // blk,),
        in_specs=[
            pl.BlockSpec((blk, H), lambda b: (b, 0)),
            pl.BlockSpec((blk, H), lambda b: (b, 0)),
            pl.BlockSpec((2 * H, H), lambda b: (0, 0)),
            pl.BlockSpec((1, H), lambda b: (0, 0)),
            pl.BlockSpec((H, H), lambda b: (0, 0)),
            pl.BlockSpec((1, H), lambda b: (0, 0)),
            pl.BlockSpec((H, 16), lambda b: (0, 0)),
            pl.BlockSpec((1, 16), lambda b: (0, 0)),
            pl.BlockSpec((16, 8), lambda b: (0, 0)),
            pl.BlockSpec((1, 8), lambda b: (0, 0)),
            pl.BlockSpec((8, 4), lambda b: (0, 0)),
            pl.BlockSpec((1, 4), lambda b: (0, 0)),
        ],
        out_specs=pl.BlockSpec((1, 4), lambda b: (0, 0)),
        out_shape=jax.ShapeDtypeStruct((1, 4), _f32),
        scratch_shapes=[pltpu.VMEM((8, H), _f32)],
    )(gi, gj, w1_t, b1.reshape(1, H), w2_t, b2.reshape(1, H), wd1_t,
      bd1.reshape(1, 16), wd2_t, bd2.reshape(1, 8), wd3_t, bd3.reshape(1, 4))


def _make_gather_body(feat, blk, nblk):
    def body(h_hbm, dst_hbm, src_hbm, gi_hbm, gj_hbm, di, si, pv, qv,
             isem, gsem, osem):
        wid = lax.axis_index("s") * NC + lax.axis_index("c")
        cbase = wid * G_CHUNK

        def issue_idx(b, slot):
            pltpu.async_copy(dst_hbm.at[pl.ds(cbase + b * blk, blk)],
                             di.at[slot], isem.at[slot])
            pltpu.async_copy(src_hbm.at[pl.ds(cbase + b * blk, blk)],
                             si.at[slot], isem.at[slot])

        def wait_idx(slot):
            pltpu.make_async_copy(dst_hbm.at[pl.ds(0, blk)], di.at[slot],
                                  isem.at[slot]).wait()
            pltpu.make_async_copy(src_hbm.at[pl.ds(0, blk)], si.at[slot],
                                  isem.at[slot]).wait()

        def issue_gather(slot):
            pltpu.async_copy(h_hbm.at[di.at[slot]], pv.at[slot], gsem.at[slot])
            pltpu.async_copy(h_hbm.at[si.at[slot]], qv.at[slot], gsem.at[slot])

        def wait_gather(slot):
            pltpu.make_async_copy(h_hbm.at[pl.ds(0, blk)], pv.at[slot],
                                  gsem.at[slot]).wait()
            pltpu.make_async_copy(h_hbm.at[pl.ds(0, blk)], qv.at[slot],
                                  gsem.at[slot]).wait()

        def wait_out(slot):
            pltpu.make_async_copy(pv.at[slot], gi_hbm.at[pl.ds(0, blk)],
                                  osem.at[slot]).wait()
            pltpu.make_async_copy(qv.at[slot], gj_hbm.at[pl.ds(0, blk)],
                                  osem.at[slot]).wait()

        issue_idx(0, 0)
        issue_idx(1, 1)
        wait_idx(0)
        issue_gather(0)

        def blkfn(b, _):
            p = lax.rem(b, 2)
            wait_gather(p)
            pltpu.async_copy(pv.at[p], gi_hbm.at[pl.ds(cbase + b * blk, blk)],
                             osem.at[p])
            pltpu.async_copy(qv.at[p], gj_hbm.at[pl.ds(cbase + b * blk, blk)],
                             osem.at[p])

            @pl.when(b + 2 < nblk)
            def _():
                issue_idx(b + 2, p)

            @pl.when(b + 1 < nblk)
            def _():
                wait_idx(1 - p)

                @pl.when(b >= 1)
                def _():
                    wait_out(1 - p)

                issue_gather(1 - p)

            return 0

        lax.fori_loop(0, nblk, blkfn, 0)
        wait_out((nblk - 2) % 2)
        wait_out((nblk - 1) % 2)

    return body


def _sc_gather(h, dst, src, feat, blk):
    nblk = G_CHUNK // blk
    mesh = plsc.VectorSubcoreMesh(core_axis_name="c", subcore_axis_name="s")
    f = pl.kernel(
        _make_gather_body(feat, blk, nblk),
        out_type=[
            jax.ShapeDtypeStruct((E, feat), _f32),
            jax.ShapeDtypeStruct((E, feat), _f32),
        ],
        mesh=mesh,
        compiler_params=_SC_PARAMS,
        scratch_types=[
            pltpu.VMEM((2, blk), _i32),
            pltpu.VMEM((2, blk), _i32),
            pltpu.VMEM((2, blk, feat), _f32),
            pltpu.VMEM((2, blk, feat), _f32),
            pltpu.SemaphoreType.DMA((2,)),
            pltpu.SemaphoreType.DMA((2,)),
            pltpu.SemaphoreType.DMA((2,)),
        ],
    )
    return f(h, dst, src)


def _sc_scatter_body(msg_hbm, dst_hbm, parts_hbm, acc, dstb, mv, tmp,
                     dsem, msem):
    wid = lax.axis_index("s") * NC + lax.axis_index("c")
    c = wid // 4
    nh = (wid // 2) % 2
    fh = wid % 2
    lane = lax.iota(_i32, 16)
    zeros = jnp.zeros((16,), _f32)

    def issue(b, slot):
        base = c * S_CHUNK + b * S_BLK
        pltpu.async_copy(dst_hbm.at[pl.ds(base, S_BLK)], dstb.at[slot],
                         dsem.at[slot])
        pltpu.async_copy(msg_hbm.at[pl.ds(base, S_BLK)], mv.at[slot],
                         msem.at[slot])

    def wait(slot):
        pltpu.make_async_copy(dst_hbm.at[pl.ds(0, S_BLK)], dstb.at[slot],
                              dsem.at[slot]).wait()
        pltpu.make_async_copy(msg_hbm.at[pl.ds(0, S_BLK)], mv.at[slot],
                              msem.at[slot]).wait()

    issue(0, 0)

    def zr(r, _):
        acc[r, pl.ds(0, 16)] = zeros
        return 0

    lax.fori_loop(0, NH_SZ, zr, 0)

    def blk(b, _):
        p = lax.rem(b, 2)

        @pl.when(b + 1 < S_NBLK)
        def _():
            issue(b + 1, 1 - p)

        wait(p)

        def grp(g, _):
            d16 = dstb[p, pl.ds(g * 16, 16)]
            local = d16 - nh * NH_SZ
            m0 = (local >= 0) & (local < NH_SZ)
            localc = jnp.where(m0, local, 0)
            erow = g * 16 + lane

            plsc.store_scatter(tmp, [localc], lane, mask=m0)
            back = plsc.load_gather(tmp, [localc], mask=m0)
            hasdup = jnp.any(m0 & (back != lane))

            for fl in range(FH_SZ):
                fcol = jnp.full((16,), fl, _i32)
                gcol = jnp.full((16,), fh * FH_SZ + fl, _i32)
                val = plsc.load_gather(mv.at[p], [erow, gcol])
                cur = plsc.load_gather(acc, [localc, fcol], mask=m0)
                new = jnp.maximum(cur, val)
                plsc.store_scatter(acc, [localc, fcol], new, mask=m0)

            @pl.when(hasdup)
            def _():
                for j in range(16):
                    mj = lane == j
                    lj = jnp.sum(jnp.where(mj, localc, 0))
                    active = jnp.any(mj & m0)

                    @pl.when(active)
                    def _():
                        er = g * 16 + j
                        sl = pl.ds(0, 16)
                        msl = pl.ds(fh * FH_SZ, 16)
                        acc[lj, sl] = jnp.maximum(acc[lj, sl], mv[p, er, msl])

            return 0

        lax.fori_loop(0, S_BLK // 16, grp, 0)
        return 0

    lax.fori_loop(0, S_NBLK, blk, 0)
    pltpu.sync_copy(
        acc, parts_hbm.at[c, pl.ds(nh * NH_SZ, NH_SZ), pl.ds(fh * FH_SZ, FH_SZ)])


def _sc_scatter(msg, dst):
    mesh = plsc.VectorSubcoreMesh(core_axis_name="c", subcore_axis_name="s")
    f = pl.kernel(
        _sc_scatter_body,
        out_type=jax.ShapeDtypeStruct((S_EC, N, H), _f32),
        mesh=mesh,
        compiler_params=_SC_PARAMS,
        scratch_types=[
            pltpu.VMEM((NH_SZ, FH_SZ), _f32),
            pltpu.VMEM((2, S_BLK), _i32),
            pltpu.VMEM((2, S_BLK, H), _f32),
            pltpu.VMEM((NH_SZ,), _i32),
            pltpu.SemaphoreType.DMA((2,)),
            pltpu.SemaphoreType.DMA((2,)),
        ],
    )
    return f(msg, dst)


def kernel(x, edge_index, edge_attr, We1, be1, We2, be2, Wc01, bc01, Wc02,
           bc02, Wc11, bc11, Wc12, bc12, Wd1, bd1, Wd2, bd2, Wd3, bd3):
    src = edge_index[0]
    dst = edge_index[1]

    gi1, gj1 = _sc_gather(x, dst, src, F_IN, 200)
    msg1 = _tc_msg(gi1, gj1, We1.T, be1, We2.T, be2, F_IN)
    parts1 = _sc_scatter(msg1, dst)
    h1 = _tc_combine(parts1)

    gi2, gj2 = _sc_gather(h1, dst, src, H, 400)
    msg2 = _tc_msg(gi2, gj2, Wc01.T, bc01, Wc02.T, bc02, H)
    parts2 = _sc_scatter(msg2, dst)
    h2 = _tc_combine(parts2)

    gi3, gj3 = _sc_gather(h2, dst, src, H, 400)
    probs = _tc_tail(gi3, gj3, Wc11.T, bc11, Wc12.T, bc12, Wd1.T, bd1,
                     Wd2.T, bd2, Wd3.T, bd3)
    return (probs, edge_attr)


# speedup vs baseline: 2.7829x; 1.2205x over previous
import jax
import jax.numpy as jnp
from jax import lax
from jax.experimental import pallas as pl
from jax.experimental.pallas import tpu as pltpu
from jax.experimental.pallas import tpu_sc as plsc

N = 10000
E = 320000
H = 32
F_IN = 128

NC = 2
NS = 16
NW = NC * NS

G_CHUNK = E // NW

S_EC = 8
S_CHUNK = E // S_EC
S_BLK = 400
S_NBLK = S_CHUNK // S_BLK
NH_SZ = N // 2
FH_SZ = H // 2

_f32 = jnp.float32
_i32 = jnp.int32

_SC_PARAMS = pltpu.CompilerParams(use_tc_tiling_on_sc=False,
                                  needs_layout_passes=False)


def _combine_body(parts_ref, h_ref):
    h_ref[...] = jnp.max(parts_ref[...], axis=0)


def _tc_combine(parts):
    blk = 1000
    return pl.pallas_call(
        _combine_body,
        grid=(N // blk,),
        in_specs=[pl.BlockSpec((S_EC, blk, H), lambda b: (0, b, 0))],
        out_specs=pl.BlockSpec((blk, H), lambda b: (b, 0)),
        out_shape=jax.ShapeDtypeStruct((N, H), _f32),
    )(parts)


def _edge_mlp(gi, gj, w1_ref, b1_ref, w2_ref, b2_ref):
    m = jnp.concatenate([gi, gj - gi], axis=-1)
    t = jax.nn.relu(jnp.dot(m, w1_ref[...], preferred_element_type=_f32)
                    + b1_ref[...])
    msg = jnp.dot(t, w2_ref[...], preferred_element_type=_f32) + b2_ref[...]
    return jnp.maximum(msg, 0.0)


def _msg_body(gi_ref, gj_ref, w1_ref, b1_ref, w2_ref, b2_ref, o_ref):
    o_ref[...] = _edge_mlp(gi_ref[...], gj_ref[...], w1_ref, b1_ref, w2_ref,
                           b2_ref)


def _tc_msg(gi, gj, w1_t, b1, w2_t, b2, f_in):
    blk = 2000
    return pl.pallas_call(
        _msg_body,
        grid=(E // blk,),
        in_specs=[
            pl.BlockSpec((blk, f_in), lambda b: (b, 0)),
            pl.BlockSpec((blk, f_in), lambda b: (b, 0)),
            pl.BlockSpec((2 * f_in, H), lambda b: (0, 0)),
            pl.BlockSpec((1, H), lambda b: (0, 0)),
            pl.BlockSpec((H, H), lambda b: (0, 0)),
            pl.BlockSpec((1, H), lambda b: (0, 0)),
        ],
        out_specs=pl.BlockSpec((blk, H), lambda b: (b, 0)),
        out_shape=jax.ShapeDtypeStruct((E, H), _f32),
    )(gi, gj, w1_t, b1.reshape(1, H), w2_t, b2.reshape(1, H))


def _tail_body(gi_ref, gj_ref, w1_ref, b1_ref, w2_ref, b2_ref, wd1_ref,
               bd1_ref, wd2_ref, bd2_ref, wd3_ref, bd3_ref, o_ref, zacc):
    i = pl.program_id(0)
    nsteps = pl.num_programs(0)
    m = _edge_mlp(gi_ref[...], gj_ref[...], w1_ref, b1_ref, w2_ref, b2_ref)
    bm = jnp.max(m.reshape(-1, 8, H), axis=0)

    @pl.when(i == 0)
    def _():
        zacc[...] = bm

    @pl.when(i > 0)
    def _():
        zacc[...] = jnp.maximum(zacc[...], bm)

    @pl.when(i == nsteps - 1)
    def _():
        z = jnp.max(zacc[...], axis=0, keepdims=True)
        d = jax.nn.relu(jnp.dot(z, wd1_ref[...], preferred_element_type=_f32)
                        + bd1_ref[...])
        d = jax.nn.relu(jnp.dot(d, wd2_ref[...], preferred_element_type=_f32)
                        + bd2_ref[...])
        o_ref[...] = (jnp.dot(d, wd3_ref[...], preferred_element_type=_f32)
                      + bd3_ref[...])


def _tc_tail(gi, gj, w1_t, b1, w2_t, b2, wd1_t, bd1, wd2_t, bd2, wd3_t, bd3):
    blk = 2000
    return pl.pallas_call(
        _tail_body,
        grid=(E // blk,),
        in_specs=[
            pl.BlockSpec((blk, H), lambda b: (b, 0)),
            pl.BlockSpec((blk, H), lambda b: (b, 0)),
            pl.BlockSpec((2 * H, H), lambda b: (0, 0)),
            pl.BlockSpec((1, H), lambda b: (0, 0)),
            pl.BlockSpec((H, H), lambda b: (0, 0)),
            pl.BlockSpec((1, H), lambda b: (0, 0)),
            pl.BlockSpec((H, 16), lambda b: (0, 0)),
            pl.BlockSpec((1, 16), lambda b: (0, 0)),
            pl.BlockSpec((16, 8), lambda b: (0, 0)),
            pl.BlockSpec((1, 8), lambda b: (0, 0)),
            pl.BlockSpec((8, 4), lambda b: (0, 0)),
            pl.BlockSpec((1, 4), lambda b: (0, 0)),
        ],
        out_specs=pl.BlockSpec((1, 4), lambda b: (0, 0)),
        out_shape=jax.ShapeDtypeStruct((1, 4), _f32),
        scratch_shapes=[pltpu.VMEM((8, H), _f32)],
    )(gi, gj, w1_t, b1.reshape(1, H), w2_t, b2.reshape(1, H), wd1_t,
      bd1.reshape(1, 16), wd2_t, bd2.reshape(1, 8), wd3_t, bd3.reshape(1, 4))


def _make_gather_body(feat, blk, nblk):
    def body(h_hbm, dst_hbm, src_hbm, gi_hbm, gj_hbm, di, si, pv, qv,
             isem, gsem, osem):
        wid = lax.axis_index("s") * NC + lax.axis_index("c")
        cbase = wid * G_CHUNK

        def issue_idx(b, slot):
            pltpu.async_copy(dst_hbm.at[pl.ds(cbase + b * blk, blk)],
                             di.at[slot], isem.at[slot])
            pltpu.async_copy(src_hbm.at[pl.ds(cbase + b * blk, blk)],
                             si.at[slot], isem.at[slot])

        def wait_idx(slot):
            pltpu.make_async_copy(dst_hbm.at[pl.ds(0, blk)], di.at[slot],
                                  isem.at[slot]).wait()
            pltpu.make_async_copy(src_hbm.at[pl.ds(0, blk)], si.at[slot],
                                  isem.at[slot]).wait()

        def issue_gather(slot):
            pltpu.async_copy(h_hbm.at[di.at[slot]], pv.at[slot], gsem.at[slot])
            pltpu.async_copy(h_hbm.at[si.at[slot]], qv.at[slot], gsem.at[slot])

        def wait_gather(slot):
            pltpu.make_async_copy(h_hbm.at[pl.ds(0, blk)], pv.at[slot],
                                  gsem.at[slot]).wait()
            pltpu.make_async_copy(h_hbm.at[pl.ds(0, blk)], qv.at[slot],
                                  gsem.at[slot]).wait()

        def wait_out(slot):
            pltpu.make_async_copy(pv.at[slot], gi_hbm.at[pl.ds(0, blk)],
                                  osem.at[slot]).wait()
            pltpu.make_async_copy(qv.at[slot], gj_hbm.at[pl.ds(0, blk)],
                                  osem.at[slot]).wait()

        issue_idx(0, 0)
        issue_idx(1, 1)
        wait_idx(0)
        issue_gather(0)

        def blkfn(b, _):
            p = lax.rem(b, 2)
            wait_gather(p)
            pltpu.async_copy(pv.at[p], gi_hbm.at[pl.ds(cbase + b * blk, blk)],
                             osem.at[p])
            pltpu.async_copy(qv.at[p], gj_hbm.at[pl.ds(cbase + b * blk, blk)],
                             osem.at[p])

            @pl.when(b + 2 < nblk)
            def _():
                issue_idx(b + 2, p)

            @pl.when(b + 1 < nblk)
            def _():
                wait_idx(1 - p)

                @pl.when(b >= 1)
                def _():
                    wait_out(1 - p)

                issue_gather(1 - p)

            return 0

        lax.fori_loop(0, nblk, blkfn, 0)
        wait_out((nblk - 2) % 2)
        wait_out((nblk - 1) % 2)

    return body


def _sc_gather(h, dst, src, feat, blk):
    nblk = G_CHUNK // blk
    mesh = plsc.VectorSubcoreMesh(core_axis_name="c", subcore_axis_name="s")
    f = pl.kernel(
        _make_gather_body(feat, blk, nblk),
        out_type=[
            jax.ShapeDtypeStruct((E, feat), _f32),
            jax.ShapeDtypeStruct((E, feat), _f32),
        ],
        mesh=mesh,
        compiler_params=_SC_PARAMS,
        scratch_types=[
            pltpu.VMEM((2, blk), _i32),
            pltpu.VMEM((2, blk), _i32),
            pltpu.VMEM((2, blk, feat), _f32),
            pltpu.VMEM((2, blk, feat), _f32),
            pltpu.SemaphoreType.DMA((2,)),
            pltpu.SemaphoreType.DMA((2,)),
            pltpu.SemaphoreType.DMA((2,)),
        ],
    )
    return f(h, dst, src)


def _sc_scatter_body(msg_hbm, dst_hbm, parts_hbm, acc, dstb, mv, tmp,
                     dsem, msem):
    wid = lax.axis_index("s") * NC + lax.axis_index("c")
    c = wid // 4
    nh = (wid // 2) % 2
    fh = wid % 2
    lane = lax.iota(_i32, 16)
    zeros = jnp.zeros((16,), _f32)

    def issue(b, slot):
        base = c * S_CHUNK + b * S_BLK
        pltpu.async_copy(dst_hbm.at[pl.ds(base, S_BLK)], dstb.at[slot],
                         dsem.at[slot])
        pltpu.async_copy(msg_hbm.at[pl.ds(base, S_BLK)], mv.at[slot],
                         msem.at[slot])

    def wait(slot):
        pltpu.make_async_copy(dst_hbm.at[pl.ds(0, S_BLK)], dstb.at[slot],
                              dsem.at[slot]).wait()
        pltpu.make_async_copy(msg_hbm.at[pl.ds(0, S_BLK)], mv.at[slot],
                              msem.at[slot]).wait()

    issue(0, 0)

    def zr(r, _):
        acc[r, pl.ds(0, 16)] = zeros
        return 0

    lax.fori_loop(0, NH_SZ, zr, 0)

    def blk(b, _):
        p = lax.rem(b, 2)

        @pl.when(b + 1 < S_NBLK)
        def _():
            issue(b + 1, 1 - p)

        wait(p)

        def grp(g, _):
            d16 = dstb[p, pl.ds(g * 16, 16)]
            local = d16 - nh * NH_SZ
            m0 = (local >= 0) & (local < NH_SZ)
            localc = jnp.where(m0, local, 0)
            erow = g * 16 + lane

            plsc.store_scatter(tmp, [localc], lane, mask=m0)
            back = plsc.load_gather(tmp, [localc], mask=m0)
            hasdup = jnp.any(m0 & (back != lane))

            news = []
            for fl in range(FH_SZ):
                fcol = jnp.full((16,), fl, _i32)
                gcol = jnp.full((16,), fh * FH_SZ + fl, _i32)
                val = plsc.load_gather(mv.at[p], [erow, gcol])
                cur = plsc.load_gather(acc, [localc, fcol], mask=m0)
                news.append(jnp.maximum(cur, val))
            for fl in range(FH_SZ):
                fcol = jnp.full((16,), fl, _i32)
                plsc.store_scatter(acc, [localc, fcol], news[fl], mask=m0)

            @pl.when(hasdup)
            def _():
                for j in range(16):
                    mj = lane == j
                    lj = jnp.sum(jnp.where(mj, localc, 0))
                    active = jnp.any(mj & m0)

                    @pl.when(active)
                    def _():
                        er = g * 16 + j
                        sl = pl.ds(0, 16)
                        msl = pl.ds(fh * FH_SZ, 16)
                        acc[lj, sl] = jnp.maximum(acc[lj, sl], mv[p, er, msl])

            return 0

        lax.fori_loop(0, S_BLK // 16, grp, 0)
        return 0

    lax.fori_loop(0, S_NBLK, blk, 0)
    pltpu.sync_copy(
        acc, parts_hbm.at[c, pl.ds(nh * NH_SZ, NH_SZ), pl.ds(fh * FH_SZ, FH_SZ)])


def _sc_scatter(msg, dst):
    mesh = plsc.VectorSubcoreMesh(core_axis_name="c", subcore_axis_name="s")
    f = pl.kernel(
        _sc_scatter_body,
        out_type=jax.ShapeDtypeStruct((S_EC, N, H), _f32),
        mesh=mesh,
        compiler_params=_SC_PARAMS,
        scratch_types=[
            pltpu.VMEM((NH_SZ, FH_SZ), _f32),
            pltpu.VMEM((2, S_BLK), _i32),
            pltpu.VMEM((2, S_BLK, H), _f32),
            pltpu.VMEM((NH_SZ,), _i32),
            pltpu.SemaphoreType.DMA((2,)),
            pltpu.SemaphoreType.DMA((2,)),
        ],
    )
    return f(msg, dst)


def kernel(x, edge_index, edge_attr, We1, be1, We2, be2, Wc01, bc01, Wc02,
           bc02, Wc11, bc11, Wc12, bc12, Wd1, bd1, Wd2, bd2, Wd3, bd3):
    src = edge_index[0]
    dst = edge_index[1]

    gi1, gj1 = _sc_gather(x, dst, src, F_IN, 200)
    msg1 = _tc_msg(gi1, gj1, We1.T, be1, We2.T, be2, F_IN)
    parts1 = _sc_scatter(msg1, dst)
    h1 = _tc_combine(parts1)

    gi2, gj2 = _sc_gather(h1, dst, src, H, 400)
    msg2 = _tc_msg(gi2, gj2, Wc01.T, bc01, Wc02.T, bc02, H)
    parts2 = _sc_scatter(msg2, dst)
    h2 = _tc_combine(parts2)

    gi3, gj3 = _sc_gather(h2, dst, src, H, 400)
    probs = _tc_tail(gi3, gj3, Wc11.T, bc11, Wc12.T, bc12, Wd1.T, bd1,
                     Wd2.T, bd2, Wd3.T, bd3)
    return (probs, edge_attr)


# speedup vs baseline: 2.8780x; 1.0342x over previous
import jax
import jax.numpy as jnp
from jax import lax
from jax.experimental import pallas as pl
from jax.experimental.pallas import tpu as pltpu
from jax.experimental.pallas import tpu_sc as plsc

N = 10000
E = 320000
H = 32
F_IN = 128

NC = 2
NS = 16
NW = NC * NS

G_CHUNK = E // NW

S_EC = 8
S_CHUNK = E // S_EC
S_BLK = 400
S_NBLK = S_CHUNK // S_BLK
NH_SZ = N // 2
FH_SZ = H // 2

_f32 = jnp.float32
_i32 = jnp.int32

_SC_PARAMS = pltpu.CompilerParams(use_tc_tiling_on_sc=False,
                                  needs_layout_passes=False)


def _combine_body(parts_ref, h_ref):
    h_ref[...] = jnp.max(parts_ref[...], axis=0)


def _tc_combine(parts):
    blk = 1000
    return pl.pallas_call(
        _combine_body,
        grid=(N // blk,),
        in_specs=[pl.BlockSpec((S_EC, blk, H), lambda b: (0, b, 0))],
        out_specs=pl.BlockSpec((blk, H), lambda b: (b, 0)),
        out_shape=jax.ShapeDtypeStruct((N, H), _f32),
    )(parts)


def _edge_mlp(gi, gj, w1_ref, b1_ref, w2_ref, b2_ref):
    m = jnp.concatenate([gi, gj - gi], axis=-1)
    t = jax.nn.relu(jnp.dot(m, w1_ref[...], preferred_element_type=_f32)
                    + b1_ref[...])
    msg = jnp.dot(t, w2_ref[...], preferred_element_type=_f32) + b2_ref[...]
    return jnp.maximum(msg, 0.0)


def _msg_body(gi_ref, gj_ref, w1_ref, b1_ref, w2_ref, b2_ref, o_ref):
    o_ref[...] = _edge_mlp(gi_ref[...], gj_ref[...], w1_ref, b1_ref, w2_ref,
                           b2_ref)


def _tc_msg(gi, gj, w1_t, b1, w2_t, b2, f_in):
    blk = 2000
    return pl.pallas_call(
        _msg_body,
        grid=(E // blk,),
        in_specs=[
            pl.BlockSpec((blk, f_in), lambda b: (b, 0)),
            pl.BlockSpec((blk, f_in), lambda b: (b, 0)),
            pl.BlockSpec((2 * f_in, H), lambda b: (0, 0)),
            pl.BlockSpec((1, H), lambda b: (0, 0)),
            pl.BlockSpec((H, H), lambda b: (0, 0)),
            pl.BlockSpec((1, H), lambda b: (0, 0)),
        ],
        out_specs=pl.BlockSpec((blk, H), lambda b: (b, 0)),
        out_shape=jax.ShapeDtypeStruct((E, H), _f32),
    )(gi, gj, w1_t, b1.reshape(1, H), w2_t, b2.reshape(1, H))


def _tail_body(gi_ref, gj_ref, w1_ref, b1_ref, w2_ref, b2_ref, wd1_ref,
               bd1_ref, wd2_ref, bd2_ref, wd3_ref, bd3_ref, o_ref, zacc):
    i = pl.program_id(0)
    nsteps = pl.num_programs(0)
    m = _edge_mlp(gi_ref[...], gj_ref[...], w1_ref, b1_ref, w2_ref, b2_ref)
    bm = jnp.max(m.reshape(-1, 8, H), axis=0)

    @pl.when(i == 0)
    def _():
        zacc[...] = bm

    @pl.when(i > 0)
    def _():
        zacc[...] = jnp.maximum(zacc[...], bm)

    @pl.when(i == nsteps - 1)
    def _():
        z = jnp.max(zacc[...], axis=0, keepdims=True)
        d = jax.nn.relu(jnp.dot(z, wd1_ref[...], preferred_element_type=_f32)
                        + bd1_ref[...])
        d = jax.nn.relu(jnp.dot(d, wd2_ref[...], preferred_element_type=_f32)
                        + bd2_ref[...])
        o_ref[...] = (jnp.dot(d, wd3_ref[...], preferred_element_type=_f32)
                      + bd3_ref[...])


def _tc_tail(gi, gj, w1_t, b1, w2_t, b2, wd1_t, bd1, wd2_t, bd2, wd3_t, bd3):
    blk = 2000
    return pl.pallas_call(
        _tail_body,
        grid=(E // blk,),
        in_specs=[
            pl.BlockSpec((blk, H), lambda b: (b, 0)),
            pl.BlockSpec((blk, H), lambda b: (b, 0)),
            pl.BlockSpec((2 * H, H), lambda b: (0, 0)),
            pl.BlockSpec((1, H), lambda b: (0, 0)),
            pl.BlockSpec((H, H), lambda b: (0, 0)),
            pl.BlockSpec((1, H), lambda b: (0, 0)),
            pl.BlockSpec((H, 16), lambda b: (0, 0)),
            pl.BlockSpec((1, 16), lambda b: (0, 0)),
            pl.BlockSpec((16, 8), lambda b: (0, 0)),
            pl.BlockSpec((1, 8), lambda b: (0, 0)),
            pl.BlockSpec((8, 4), lambda b: (0, 0)),
            pl.BlockSpec((1, 4), lambda b: (0, 0)),
        ],
        out_specs=pl.BlockSpec((1, 4), lambda b: (0, 0)),
        out_shape=jax.ShapeDtypeStruct((1, 4), _f32),
        scratch_shapes=[pltpu.VMEM((8, H), _f32)],
    )(gi, gj, w1_t, b1.reshape(1, H), w2_t, b2.reshape(1, H), wd1_t,
      bd1.reshape(1, 16), wd2_t, bd2.reshape(1, 8), wd3_t, bd3.reshape(1, 4))


def _make_gather_body(feat, blk, nblk):
    def body(h_hbm, dst_hbm, src_hbm, gi_hbm, gj_hbm, di, si, pv, qv,
             isem, gsem, osem):
        wid = lax.axis_index("s") * NC + lax.axis_index("c")
        cbase = wid * G_CHUNK

        def issue_idx(b, slot):
            pltpu.async_copy(dst_hbm.at[pl.ds(cbase + b * blk, blk)],
                             di.at[slot], isem.at[slot])
            pltpu.async_copy(src_hbm.at[pl.ds(cbase + b * blk, blk)],
                             si.at[slot], isem.at[slot])

        def wait_idx(slot):
            pltpu.make_async_copy(dst_hbm.at[pl.ds(0, blk)], di.at[slot],
                                  isem.at[slot]).wait()
            pltpu.make_async_copy(src_hbm.at[pl.ds(0, blk)], si.at[slot],
                                  isem.at[slot]).wait()

        def issue_gather(slot):
            pltpu.async_copy(h_hbm.at[di.at[slot]], pv.at[slot], gsem.at[slot])
            pltpu.async_copy(h_hbm.at[si.at[slot]], qv.at[slot], gsem.at[slot])

        def wait_gather(slot):
            pltpu.make_async_copy(h_hbm.at[pl.ds(0, blk)], pv.at[slot],
                                  gsem.at[slot]).wait()
            pltpu.make_async_copy(h_hbm.at[pl.ds(0, blk)], qv.at[slot],
                                  gsem.at[slot]).wait()

        def wait_out(slot):
            pltpu.make_async_copy(pv.at[slot], gi_hbm.at[pl.ds(0, blk)],
                                  osem.at[slot]).wait()
            pltpu.make_async_copy(qv.at[slot], gj_hbm.at[pl.ds(0, blk)],
                                  osem.at[slot]).wait()

        issue_idx(0, 0)
        issue_idx(1, 1)
        wait_idx(0)
        issue_gather(0)

        def blkfn(b, _):
            p = lax.rem(b, 2)
            wait_gather(p)
            pltpu.async_copy(pv.at[p], gi_hbm.at[pl.ds(cbase + b * blk, blk)],
                             osem.at[p])
            pltpu.async_copy(qv.at[p], gj_hbm.at[pl.ds(cbase + b * blk, blk)],
                             osem.at[p])

            @pl.when(b + 2 < nblk)
            def _():
                issue_idx(b + 2, p)

            @pl.when(b + 1 < nblk)
            def _():
                wait_idx(1 - p)

                @pl.when(b >= 1)
                def _():
                    wait_out(1 - p)

                issue_gather(1 - p)

            return 0

        lax.fori_loop(0, nblk, blkfn, 0)
        wait_out((nblk - 2) % 2)
        wait_out((nblk - 1) % 2)

    return body


def _sc_gather(h, dst, src, feat, blk):
    nblk = G_CHUNK // blk
    mesh = plsc.VectorSubcoreMesh(core_axis_name="c", subcore_axis_name="s")
    f = pl.kernel(
        _make_gather_body(feat, blk, nblk),
        out_type=[
            jax.ShapeDtypeStruct((E, feat), _f32),
            jax.ShapeDtypeStruct((E, feat), _f32),
        ],
        mesh=mesh,
        compiler_params=_SC_PARAMS,
        scratch_types=[
            pltpu.VMEM((2, blk), _i32),
            pltpu.VMEM((2, blk), _i32),
            pltpu.VMEM((2, blk, feat), _f32),
            pltpu.VMEM((2, blk, feat), _f32),
            pltpu.SemaphoreType.DMA((2,)),
            pltpu.SemaphoreType.DMA((2,)),
            pltpu.SemaphoreType.DMA((2,)),
        ],
    )
    return f(h, dst, src)


def _sc_scatter_body(msg_hbm, dst_hbm, parts_hbm, acc, dstb, mv, tmp,
                     cloc, crow, dsem, msem):
    wid = lax.axis_index("s") * NC + lax.axis_index("c")
    c = wid // 4
    nh = (wid // 2) % 2
    fh = wid % 2
    lane = lax.iota(_i32, 16)
    zeros = jnp.zeros((16,), _f32)

    def issue(b, slot):
        base = c * S_CHUNK + b * S_BLK
        pltpu.async_copy(dst_hbm.at[pl.ds(base, S_BLK)], dstb.at[slot],
                         dsem.at[slot])
        pltpu.async_copy(msg_hbm.at[pl.ds(base, S_BLK)], mv.at[slot],
                         msem.at[slot])

    def wait(slot):
        pltpu.make_async_copy(dst_hbm.at[pl.ds(0, S_BLK)], dstb.at[slot],
                              dsem.at[slot]).wait()
        pltpu.make_async_copy(msg_hbm.at[pl.ds(0, S_BLK)], mv.at[slot],
                              msem.at[slot]).wait()

    issue(0, 0)

    def zr(r, _):
        acc[r, pl.ds(0, 16)] = zeros
        return 0

    lax.fori_loop(0, NH_SZ, zr, 0)

    def blk(b, _):
        p = lax.rem(b, 2)

        @pl.when(b + 1 < S_NBLK)
        def _():
            issue(b + 1, 1 - p)

        wait(p)

        def scan(g, off):
            d16 = dstb[p, pl.ds(g * 16, 16)]
            local = d16 - nh * NH_SZ
            m0 = (local >= 0) & (local < NH_SZ)
            localc = jnp.where(m0, local, 0)
            erow = g * 16 + lane
            plsc.store_compressed(cloc.at[pl.ds(off, 16)], localc, mask=m0)
            plsc.store_compressed(crow.at[pl.ds(off, 16)], erow, mask=m0)
            return off + jnp.sum(m0.astype(_i32))

        cnt = lax.fori_loop(0, S_BLK // 16, scan, 0)

        def rmw(r, _):
            mr = r * 16 + lane < cnt
            cl = jnp.where(mr, cloc[pl.ds(r * 16, 16)], 0)
            ce = jnp.where(mr, crow[pl.ds(r * 16, 16)], 0)

            plsc.store_scatter(tmp, [cl], lane, mask=mr)
            back = plsc.load_gather(tmp, [cl], mask=mr)
            hasdup = jnp.any(mr & (back != lane))

            news = []
            for fl in range(FH_SZ):
                fcol = jnp.full((16,), fl, _i32)
                gcol = jnp.full((16,), fh * FH_SZ + fl, _i32)
                val = plsc.load_gather(mv.at[p], [ce, gcol])
                cur = plsc.load_gather(acc, [cl, fcol], mask=mr)
                news.append(jnp.maximum(cur, val))
            for fl in range(FH_SZ):
                fcol = jnp.full((16,), fl, _i32)
                plsc.store_scatter(acc, [cl, fcol], news[fl], mask=mr)

            @pl.when(hasdup)
            def _():
                for j in range(16):
                    mj = lane == j
                    lj = jnp.sum(jnp.where(mj, cl, 0))
                    er = jnp.sum(jnp.where(mj, ce, 0))
                    active = jnp.any(mj & mr)

                    @pl.when(active)
                    def _():
                        sl = pl.ds(0, 16)
                        msl = pl.ds(fh * FH_SZ, 16)
                        acc[lj, sl] = jnp.maximum(acc[lj, sl], mv[p, er, msl])

            return 0

        lax.fori_loop(0, (cnt + 15) // 16, rmw, 0)
        return 0

    lax.fori_loop(0, S_NBLK, blk, 0)
    pltpu.sync_copy(
        acc, parts_hbm.at[c, pl.ds(nh * NH_SZ, NH_SZ), pl.ds(fh * FH_SZ, FH_SZ)])


def _sc_scatter(msg, dst):
    mesh = plsc.VectorSubcoreMesh(core_axis_name="c", subcore_axis_name="s")
    f = pl.kernel(
        _sc_scatter_body,
        out_type=jax.ShapeDtypeStruct((S_EC, N, H), _f32),
        mesh=mesh,
        compiler_params=_SC_PARAMS,
        scratch_types=[
            pltpu.VMEM((NH_SZ, FH_SZ), _f32),
            pltpu.VMEM((2, S_BLK), _i32),
            pltpu.VMEM((2, S_BLK, H), _f32),
            pltpu.VMEM((NH_SZ,), _i32),
            pltpu.VMEM((S_BLK + 112, ), _i32),
            pltpu.VMEM((S_BLK + 112, ), _i32),
            pltpu.SemaphoreType.DMA((2,)),
            pltpu.SemaphoreType.DMA((2,)),
        ],
    )
    return f(msg, dst)


def kernel(x, edge_index, edge_attr, We1, be1, We2, be2, Wc01, bc01, Wc02,
           bc02, Wc11, bc11, Wc12, bc12, Wd1, bd1, Wd2, bd2, Wd3, bd3):
    src = edge_index[0]
    dst = edge_index[1]

    gi1, gj1 = _sc_gather(x, dst, src, F_IN, 200)
    msg1 = _tc_msg(gi1, gj1, We1.T, be1, We2.T, be2, F_IN)
    parts1 = _sc_scatter(msg1, dst)
    h1 = _tc_combine(parts1)

    gi2, gj2 = _sc_gather(h1, dst, src, H, 400)
    msg2 = _tc_msg(gi2, gj2, Wc01.T, bc01, Wc02.T, bc02, H)
    parts2 = _sc_scatter(msg2, dst)
    h2 = _tc_combine(parts2)

    gi3, gj3 = _sc_gather(h2, dst, src, H, 400)
    probs = _tc_tail(gi3, gj3, Wc11.T, bc11, Wc12.T, bc12, Wd1.T, bd1,
                     Wd2.T, bd2, Wd3.T, bd3)
    return (probs, edge_attr)
